# baseline (speedup 1.0000x reference)
import functools

import jax
import jax.numpy as jnp
from jax import lax
from jax.experimental import pallas as pl
from jax.experimental.pallas import tpu as pltpu
from jax.experimental.pallas import tpu_sc as plsc

D_MODEL = 1024
N_TOK = 8192
N_EXP = 8
ROW_TILE = 256
N_PAD = N_TOK + N_EXP * ROW_TILE
N_TILES = N_PAD // ROW_TILE

_NUM_CORES = 2
_NUM_SUBCORES = 16
_NW = _NUM_CORES * _NUM_SUBCORES
_CHUNK = 32


def _sc_scatter_rows(src, idx3, n_out):
    n_in, d = src.shape
    nw, n_chunks, chunk = idx3.shape
    assert nw == _NW and n_chunks * chunk * nw == n_in
    assert n_chunks % 2 == 0

    mesh = plsc.VectorSubcoreMesh(core_axis_name="c", subcore_axis_name="s")

    @functools.partial(
        pl.kernel,
        out_type=jax.ShapeDtypeStruct((n_out, d), jnp.float32),
        mesh=mesh,
        scratch_types=[
            pltpu.VMEM((n_chunks, chunk), jnp.int32),
            pltpu.VMEM((chunk, d), jnp.float32),
            pltpu.VMEM((chunk, d), jnp.float32),
            pltpu.SemaphoreType.DMA,
            pltpu.SemaphoreType.DMA,
            pltpu.SemaphoreType.DMA,
            pltpu.SemaphoreType.DMA,
        ],
    )
    def k(src_hbm, idx_hbm, out_hbm, idx_v, buf0, buf1, l0, l1, s0, s1):
        wid = lax.axis_index("s") * _NUM_CORES + lax.axis_index("c")
        base = wid * (n_chunks * chunk)
        pltpu.sync_copy(idx_hbm.at[wid], idx_v)

        def load(c, buf, sem):
            pltpu.async_copy(src_hbm.at[pl.ds(base + c * chunk, chunk)],
                             buf, sem)

        def scatter(c, buf, sem):
            pltpu.async_copy(buf, out_hbm.at[idx_v.at[c]], sem)

        def wait_load(buf, sem):
            pltpu.make_async_copy(src_hbm.at[pl.ds(0, chunk)], buf,
                                  sem).wait()

        def wait_scatter(buf, sem):
            pltpu.make_async_copy(buf, out_hbm.at[pl.ds(0, chunk)],
                                  sem).wait()

        load(0, buf0, l0)

        def pair(p, carry):
            c0 = 2 * p
            wait_load(buf0, l0)
            scatter(c0, buf0, s0)
            load(c0 + 1, buf1, l1)
            wait_load(buf1, l1)
            scatter(c0 + 1, buf1, s1)
            wait_scatter(buf0, s0)

            @pl.when(c0 + 2 < n_chunks)
            def _():
                load(c0 + 2, buf0, l0)

            wait_scatter(buf1, s1)
            return carry

        lax.fori_loop(0, n_chunks // 2, pair, 0)

    return k(src, idx3)


def _sc_gather_rows(src, idx, n_out, chunk=_CHUNK):
    d = src.shape[1]
    rows_per_w = n_out // _NW
    n_chunks = rows_per_w // chunk
    assert rows_per_w % chunk == 0 and chunk <= 128 and chunk % 8 == 0
    assert n_chunks % 2 == 0 and n_chunks >= 2

    mesh = plsc.VectorSubcoreMesh(core_axis_name="c", subcore_axis_name="s")

    @functools.partial(
        pl.kernel,
        out_type=jax.ShapeDtypeStruct((n_out, d), jnp.float32),
        mesh=mesh,
        scratch_types=[
            pltpu.VMEM((rows_per_w,), jnp.int32),
            pltpu.VMEM((chunk, d), jnp.float32),
            pltpu.VMEM((chunk, d), jnp.float32),
            pltpu.SemaphoreType.DMA,
            pltpu.SemaphoreType.DMA,
            pltpu.SemaphoreType.DMA,
            pltpu.SemaphoreType.DMA,
        ],
    )
    def k(src_hbm, idx_hbm, out_hbm, idx_v, buf0, buf1, g0, g1, w0, w1):
        wid = lax.axis_index("s") * _NUM_CORES + lax.axis_index("c")
        base = wid * rows_per_w
        pltpu.sync_copy(idx_hbm.at[pl.ds(base, rows_per_w)], idx_v)

        def gather(c, buf, sem):
            pltpu.async_copy(src_hbm.at[idx_v.at[pl.ds(c * chunk, chunk)]],
                             buf, sem)

        def writeback(c, buf, sem):
            pltpu.async_copy(buf, out_hbm.at[pl.ds(base + c * chunk, chunk)],
                             sem)

        def wait_gather(buf, sem):
            pltpu.make_async_copy(src_hbm.at[pl.ds(0, chunk)], buf, sem).wait()

        def wait_writeback(buf, sem):
            pltpu.make_async_copy(buf, out_hbm.at[pl.ds(base, chunk)],
                                  sem).wait()

        gather(0, buf0, g0)

        def pair(p, carry):
            c0 = 2 * p
            wait_gather(buf0, g0)
            writeback(c0, buf0, w0)
            gather(c0 + 1, buf1, g1)
            wait_gather(buf1, g1)
            writeback(c0 + 1, buf1, w1)
            wait_writeback(buf0, w0)

            @pl.when(c0 + 2 < n_chunks)
            def _():
                gather(c0 + 2, buf0, g0)

            wait_writeback(buf1, w1)
            return carry

        lax.fori_loop(0, n_chunks // 2, pair, 0)

    return k(src, idx)


def _grouped_matmul_half(x_half, W, b3, tile_expert, tile_off, y_prev):
    half_tiles = N_TILES // 2

    def mm_body(te_ref, x_ref, w_ref, b_ref, y_in_ref, o_ref):
        del y_in_ref
        o_ref[...] = (
            jnp.dot(x_ref[...], w_ref[0], preferred_element_type=jnp.float32)
            + b_ref[0]
        )

    grid_spec = pltpu.PrefetchScalarGridSpec(
        num_scalar_prefetch=1,
        grid=(half_tiles,),
        in_specs=[
            pl.BlockSpec((ROW_TILE, D_MODEL), lambda t, te: (t, 0)),
            pl.BlockSpec((1, D_MODEL, D_MODEL),
                         lambda t, te: (te[t + tile_off], 0, 0)),
            pl.BlockSpec((1, 1, D_MODEL),
                         lambda t, te: (te[t + tile_off], 0, 0)),
            pl.BlockSpec((ROW_TILE, D_MODEL),
                         lambda t, te: (t + tile_off, 0)),
        ],
        out_specs=pl.BlockSpec((ROW_TILE, D_MODEL),
                               lambda t, te: (t + tile_off, 0)),
    )
    return pl.pallas_call(
        mm_body,
        grid_spec=grid_spec,
        out_shape=jax.ShapeDtypeStruct((N_PAD, D_MODEL), jnp.float32),
        input_output_aliases={4: 0},
    )(tile_expert, x_half, W, b3, y_prev)


def kernel(x, cell_dimensions, W, b):
    cd = cell_dimensions.astype(jnp.int32)

    onehot_t = (cd[None, :] == jnp.arange(N_EXP, dtype=jnp.int32)[:, None]
                ).astype(jnp.int32)
    cs = jnp.cumsum(onehot_t, axis=1)
    counts = cs[:, -1]
    padded = ((counts + ROW_TILE - 1) // ROW_TILE) * ROW_TILE
    padded_ends = jnp.cumsum(padded)
    offs_pad = padded_ends - padded
    dest_token = jnp.sum(onehot_t * (cs - 1 + offs_pad[:, None]), axis=0)
    dest_token = dest_token.astype(jnp.int32)
    tile_expert = jnp.minimum(
        (jnp.arange(N_TILES, dtype=jnp.int32)[:, None] * ROW_TILE
         >= padded_ends[None, :]).sum(1),
        N_EXP - 1).astype(jnp.int32)

    src_rows = (jnp.arange(N_PAD, dtype=jnp.int32) % N_TOK).at[
        dest_token].set(jnp.arange(N_TOK, dtype=jnp.int32))

    half = N_PAD // 2
    b3 = b.reshape(N_EXP, 1, D_MODEL)
    xpa = _sc_gather_rows(x, src_rows[:half], half, chunk=40)
    xpb = _sc_gather_rows(x, src_rows[half:], half, chunk=40)
    y0 = _grouped_matmul_half(xpa, W, b3, tile_expert, 0,
                              jnp.empty((N_PAD, D_MODEL), jnp.float32))
    y1 = _grouped_matmul_half(xpb, W, b3, tile_expert, N_TILES // 2, y0)
    out = _sc_gather_rows(y1, dest_token, N_TOK)
    return out


# speedup vs baseline: 1.2611x; 1.2611x over previous
import functools

import jax
import jax.numpy as jnp
from jax import lax
from jax.experimental import pallas as pl
from jax.experimental.pallas import tpu as pltpu
from jax.experimental.pallas import tpu_sc as plsc

D_MODEL = 1024
N_TOK = 8192
N_EXP = 8
ROW_TILE = 128
N_PAD = N_TOK + N_EXP * ROW_TILE
N_TILES = N_PAD // ROW_TILE

_NUM_CORES = 2
_NUM_SUBCORES = 16
_NW = _NUM_CORES * _NUM_SUBCORES
_CHUNK = 32


def _sc_scatter_rows(src, idx3, n_out):
    n_in, d = src.shape
    nw, n_chunks, chunk = idx3.shape
    assert nw == _NW and n_chunks * chunk * nw == n_in
    assert n_chunks % 2 == 0

    mesh = plsc.VectorSubcoreMesh(core_axis_name="c", subcore_axis_name="s")

    @functools.partial(
        pl.kernel,
        out_type=jax.ShapeDtypeStruct((n_out, d), jnp.float32),
        mesh=mesh,
        scratch_types=[
            pltpu.VMEM((n_chunks, chunk), jnp.int32),
            pltpu.VMEM((chunk, d), jnp.float32),
            pltpu.VMEM((chunk, d), jnp.float32),
            pltpu.SemaphoreType.DMA,
            pltpu.SemaphoreType.DMA,
            pltpu.SemaphoreType.DMA,
            pltpu.SemaphoreType.DMA,
        ],
    )
    def k(src_hbm, idx_hbm, out_hbm, idx_v, buf0, buf1, l0, l1, s0, s1):
        wid = lax.axis_index("s") * _NUM_CORES + lax.axis_index("c")
        base = wid * (n_chunks * chunk)
        pltpu.sync_copy(idx_hbm.at[wid], idx_v)

        def load(c, buf, sem):
            pltpu.async_copy(src_hbm.at[pl.ds(base + c * chunk, chunk)],
                             buf, sem)

        def scatter(c, buf, sem):
            pltpu.async_copy(buf, out_hbm.at[idx_v.at[c]], sem)

        def wait_load(buf, sem):
            pltpu.make_async_copy(src_hbm.at[pl.ds(0, chunk)], buf,
                                  sem).wait()

        def wait_scatter(buf, sem):
            pltpu.make_async_copy(buf, out_hbm.at[pl.ds(0, chunk)],
                                  sem).wait()

        load(0, buf0, l0)

        def pair(p, carry):
            c0 = 2 * p
            wait_load(buf0, l0)
            scatter(c0, buf0, s0)
            load(c0 + 1, buf1, l1)
            wait_load(buf1, l1)
            scatter(c0 + 1, buf1, s1)
            wait_scatter(buf0, s0)

            @pl.when(c0 + 2 < n_chunks)
            def _():
                load(c0 + 2, buf0, l0)

            wait_scatter(buf1, s1)
            return carry

        lax.fori_loop(0, n_chunks // 2, pair, 0)

    return k(src, idx3)


def _sc_gather_rows(src, idx, n_out, chunk=_CHUNK):
    d = src.shape[1]
    rows_per_w = n_out // _NW
    n_chunks = rows_per_w // chunk
    assert rows_per_w % chunk == 0 and chunk <= 128 and chunk % 8 == 0
    assert n_chunks % 2 == 0 and n_chunks >= 2

    mesh = plsc.VectorSubcoreMesh(core_axis_name="c", subcore_axis_name="s")

    @functools.partial(
        pl.kernel,
        out_type=jax.ShapeDtypeStruct((n_out, d), jnp.float32),
        mesh=mesh,
        scratch_types=[
            pltpu.VMEM((rows_per_w,), jnp.int32),
            pltpu.VMEM((chunk, d), jnp.float32),
            pltpu.VMEM((chunk, d), jnp.float32),
            pltpu.SemaphoreType.DMA,
            pltpu.SemaphoreType.DMA,
            pltpu.SemaphoreType.DMA,
            pltpu.SemaphoreType.DMA,
        ],
    )
    def k(src_hbm, idx_hbm, out_hbm, idx_v, buf0, buf1, g0, g1, w0, w1):
        wid = lax.axis_index("s") * _NUM_CORES + lax.axis_index("c")
        base = wid * rows_per_w
        pltpu.sync_copy(idx_hbm.at[pl.ds(base, rows_per_w)], idx_v)

        def gather(c, buf, sem):
            pltpu.async_copy(src_hbm.at[idx_v.at[pl.ds(c * chunk, chunk)]],
                             buf, sem)

        def writeback(c, buf, sem):
            pltpu.async_copy(buf, out_hbm.at[pl.ds(base + c * chunk, chunk)],
                             sem)

        def wait_gather(buf, sem):
            pltpu.make_async_copy(src_hbm.at[pl.ds(0, chunk)], buf, sem).wait()

        def wait_writeback(buf, sem):
            pltpu.make_async_copy(buf, out_hbm.at[pl.ds(base, chunk)],
                                  sem).wait()

        gather(0, buf0, g0)

        def pair(p, carry):
            c0 = 2 * p
            wait_gather(buf0, g0)
            writeback(c0, buf0, w0)
            gather(c0 + 1, buf1, g1)
            wait_gather(buf1, g1)
            writeback(c0 + 1, buf1, w1)
            wait_writeback(buf0, w0)

            @pl.when(c0 + 2 < n_chunks)
            def _():
                gather(c0 + 2, buf0, g0)

            wait_writeback(buf1, w1)
            return carry

        lax.fori_loop(0, n_chunks // 2, pair, 0)

    return k(src, idx)


def _grouped_matmul(x_pad, W, b, tile_expert):
    def mm_body(te_ref, x_ref, w_ref, b_ref, o_ref):
        o_ref[...] = (
            jnp.dot(x_ref[...], w_ref[0], preferred_element_type=jnp.float32)
            + b_ref[0]
        )

    grid_spec = pltpu.PrefetchScalarGridSpec(
        num_scalar_prefetch=1,
        grid=(N_TILES,),
        in_specs=[
            pl.BlockSpec((ROW_TILE, D_MODEL), lambda t, te: (t, 0)),
            pl.BlockSpec((1, D_MODEL, D_MODEL), lambda t, te: (te[t], 0, 0)),
            pl.BlockSpec((1, 1, D_MODEL), lambda t, te: (te[t], 0, 0)),
        ],
        out_specs=pl.BlockSpec((ROW_TILE, D_MODEL), lambda t, te: (t, 0)),
    )
    return pl.pallas_call(
        mm_body,
        grid_spec=grid_spec,
        out_shape=jax.ShapeDtypeStruct((N_PAD, D_MODEL), jnp.float32),
    )(tile_expert, x_pad, W, b.reshape(N_EXP, 1, D_MODEL))


def kernel(x, cell_dimensions, W, b):
    cd = cell_dimensions.astype(jnp.int32)

    onehot_t = (cd[None, :] == jnp.arange(N_EXP, dtype=jnp.int32)[:, None]
                ).astype(jnp.int32)
    cs = jnp.cumsum(onehot_t, axis=1)
    counts = cs[:, -1]
    padded = ((counts + ROW_TILE - 1) // ROW_TILE) * ROW_TILE
    padded_ends = jnp.cumsum(padded)
    offs_pad = padded_ends - padded
    dest_token = jnp.sum(onehot_t * (cs - 1 + offs_pad[:, None]), axis=0)
    dest_token = dest_token.astype(jnp.int32)
    tile_expert = jnp.minimum(
        (jnp.arange(N_TILES, dtype=jnp.int32)[:, None] * ROW_TILE
         >= padded_ends[None, :]).sum(1),
        N_EXP - 1).astype(jnp.int32)

    idx3 = dest_token.reshape(_NW, (N_TOK // _NW) // _CHUNK, _CHUNK)
    x_pad = _sc_scatter_rows(x, idx3, N_PAD)
    y_pad = _grouped_matmul(x_pad, W, b, tile_expert)
    out = _sc_gather_rows(y_pad, dest_token, N_TOK)
    return out


# speedup vs baseline: 1.4012x; 1.1111x over previous
import functools

import jax
import jax.numpy as jnp
from jax import lax
from jax.experimental import pallas as pl
from jax.experimental.pallas import tpu as pltpu
from jax.experimental.pallas import tpu_sc as plsc

D_MODEL = 1024
N_TOK = 8192
N_EXP = 8
ROW_TILE = 256
N_PAD = N_TOK + N_EXP * ROW_TILE
N_TILES = N_PAD // ROW_TILE

_NUM_CORES = 2
_NUM_SUBCORES = 16
_NW = _NUM_CORES * _NUM_SUBCORES
_CHUNK = 32


def _sc_scatter_rows(src, idx3, n_out):
    n_in, d = src.shape
    nw, n_chunks, chunk = idx3.shape
    assert nw == _NW and n_chunks * chunk * nw == n_in
    assert n_chunks % 2 == 0

    mesh = plsc.VectorSubcoreMesh(core_axis_name="c", subcore_axis_name="s")

    @functools.partial(
        pl.kernel,
        out_type=jax.ShapeDtypeStruct((n_out, d), jnp.float32),
        mesh=mesh,
        scratch_types=[
            pltpu.VMEM((n_chunks, chunk), jnp.int32),
            pltpu.VMEM((chunk, d), jnp.float32),
            pltpu.VMEM((chunk, d), jnp.float32),
            pltpu.SemaphoreType.DMA,
            pltpu.SemaphoreType.DMA,
            pltpu.SemaphoreType.DMA,
            pltpu.SemaphoreType.DMA,
        ],
    )
    def k(src_hbm, idx_hbm, out_hbm, idx_v, buf0, buf1, l0, l1, s0, s1):
        wid = lax.axis_index("s") * _NUM_CORES + lax.axis_index("c")
        base = wid * (n_chunks * chunk)
        pltpu.sync_copy(idx_hbm.at[wid], idx_v)

        def load(c, buf, sem):
            pltpu.async_copy(src_hbm.at[pl.ds(base + c * chunk, chunk)],
                             buf, sem)

        def scatter(c, buf, sem):
            pltpu.async_copy(buf, out_hbm.at[idx_v.at[c]], sem)

        def wait_load(buf, sem):
            pltpu.make_async_copy(src_hbm.at[pl.ds(0, chunk)], buf,
                                  sem).wait()

        def wait_scatter(buf, sem):
            pltpu.make_async_copy(buf, out_hbm.at[pl.ds(0, chunk)],
                                  sem).wait()

        load(0, buf0, l0)

        def pair(p, carry):
            c0 = 2 * p
            wait_load(buf0, l0)
            scatter(c0, buf0, s0)
            load(c0 + 1, buf1, l1)
            wait_load(buf1, l1)
            scatter(c0 + 1, buf1, s1)
            wait_scatter(buf0, s0)

            @pl.when(c0 + 2 < n_chunks)
            def _():
                load(c0 + 2, buf0, l0)

            wait_scatter(buf1, s1)
            return carry

        lax.fori_loop(0, n_chunks // 2, pair, 0)

    return k(src, idx3)


def _sc_gather_rows(src, idx, n_out, chunk=_CHUNK):
    d = src.shape[1]
    rows_per_w = n_out // _NW
    n_chunks = rows_per_w // chunk
    assert rows_per_w % chunk == 0 and chunk <= 128 and chunk % 8 == 0
    assert n_chunks % 2 == 0 and n_chunks >= 2

    mesh = plsc.VectorSubcoreMesh(core_axis_name="c", subcore_axis_name="s")

    @functools.partial(
        pl.kernel,
        out_type=jax.ShapeDtypeStruct((n_out, d), jnp.float32),
        mesh=mesh,
        scratch_types=[
            pltpu.VMEM((rows_per_w,), jnp.int32),
            pltpu.VMEM((chunk, d), jnp.float32),
            pltpu.VMEM((chunk, d), jnp.float32),
            pltpu.SemaphoreType.DMA,
            pltpu.SemaphoreType.DMA,
            pltpu.SemaphoreType.DMA,
            pltpu.SemaphoreType.DMA,
        ],
    )
    def k(src_hbm, idx_hbm, out_hbm, idx_v, buf0, buf1, g0, g1, w0, w1):
        wid = lax.axis_index("s") * _NUM_CORES + lax.axis_index("c")
        base = wid * rows_per_w
        pltpu.sync_copy(idx_hbm.at[pl.ds(base, rows_per_w)], idx_v)

        def gather(c, buf, sem):
            pltpu.async_copy(src_hbm.at[idx_v.at[pl.ds(c * chunk, chunk)]],
                             buf, sem)

        def writeback(c, buf, sem):
            pltpu.async_copy(buf, out_hbm.at[pl.ds(base + c * chunk, chunk)],
                             sem)

        def wait_gather(buf, sem):
            pltpu.make_async_copy(src_hbm.at[pl.ds(0, chunk)], buf, sem).wait()

        def wait_writeback(buf, sem):
            pltpu.make_async_copy(buf, out_hbm.at[pl.ds(base, chunk)],
                                  sem).wait()

        gather(0, buf0, g0)

        def pair(p, carry):
            c0 = 2 * p
            wait_gather(buf0, g0)
            writeback(c0, buf0, w0)
            gather(c0 + 1, buf1, g1)
            wait_gather(buf1, g1)
            writeback(c0 + 1, buf1, w1)
            wait_writeback(buf0, w0)

            @pl.when(c0 + 2 < n_chunks)
            def _():
                gather(c0 + 2, buf0, g0)

            wait_writeback(buf1, w1)
            return carry

        lax.fori_loop(0, n_chunks // 2, pair, 0)

    return k(src, idx)


def _grouped_matmul(x_pad, W, b, tile_expert):
    def mm_body(te_ref, x_ref, w_ref, b_ref, o_ref):
        o_ref[...] = (
            jnp.dot(x_ref[...], w_ref[0], preferred_element_type=jnp.float32)
            + b_ref[0]
        )

    grid_spec = pltpu.PrefetchScalarGridSpec(
        num_scalar_prefetch=1,
        grid=(N_TILES,),
        in_specs=[
            pl.BlockSpec((ROW_TILE, D_MODEL), lambda t, te: (t, 0)),
            pl.BlockSpec((1, D_MODEL, D_MODEL), lambda t, te: (te[t], 0, 0)),
            pl.BlockSpec((1, 1, D_MODEL), lambda t, te: (te[t], 0, 0)),
        ],
        out_specs=pl.BlockSpec((ROW_TILE, D_MODEL), lambda t, te: (t, 0)),
    )
    return pl.pallas_call(
        mm_body,
        grid_spec=grid_spec,
        out_shape=jax.ShapeDtypeStruct((N_PAD, D_MODEL), jnp.float32),
    )(tile_expert, x_pad, W, b.reshape(N_EXP, 1, D_MODEL))


def kernel(x, cell_dimensions, W, b):
    cd = cell_dimensions.astype(jnp.int32)

    onehot_t = (cd[None, :] == jnp.arange(N_EXP, dtype=jnp.int32)[:, None]
                ).astype(jnp.int32)
    cs = jnp.cumsum(onehot_t, axis=1)
    counts = cs[:, -1]
    padded = ((counts + ROW_TILE - 1) // ROW_TILE) * ROW_TILE
    padded_ends = jnp.cumsum(padded)
    offs_pad = padded_ends - padded
    dest_token = jnp.sum(onehot_t * (cs - 1 + offs_pad[:, None]), axis=0)
    dest_token = dest_token.astype(jnp.int32)
    tile_expert = jnp.minimum(
        (jnp.arange(N_TILES, dtype=jnp.int32)[:, None] * ROW_TILE
         >= padded_ends[None, :]).sum(1),
        N_EXP - 1).astype(jnp.int32)

    idx3 = dest_token.reshape(_NW, (N_TOK // _NW) // _CHUNK, _CHUNK)
    x_pad = _sc_scatter_rows(x, idx3, N_PAD)
    y_pad = _grouped_matmul(x_pad, W, b, tile_expert)
    out = _sc_gather_rows(y_pad, dest_token, N_TOK)
    return out


# speedup vs baseline: 1.4702x; 1.0492x over previous
import functools

import jax
import jax.numpy as jnp
from jax import lax
from jax.experimental import pallas as pl
from jax.experimental.pallas import tpu as pltpu
from jax.experimental.pallas import tpu_sc as plsc

D_MODEL = 1024
N_TOK = 8192
N_EXP = 8
ROW_TILE = 256
N_PAD = N_TOK + N_EXP * ROW_TILE
N_TILES = N_PAD // ROW_TILE

_NUM_CORES = 2
_NUM_SUBCORES = 16
_NW = _NUM_CORES * _NUM_SUBCORES
_CHUNK = 32


def _sc_scatter_rows(src, idx3, n_out):
    n_in, d = src.shape
    nw, n_chunks, chunk = idx3.shape
    assert nw == _NW and n_chunks * chunk * nw == n_in
    assert n_chunks % 2 == 0

    mesh = plsc.VectorSubcoreMesh(core_axis_name="c", subcore_axis_name="s")

    @functools.partial(
        pl.kernel,
        out_type=jax.ShapeDtypeStruct((n_out, d), jnp.float32),
        mesh=mesh,
        scratch_types=[
            pltpu.VMEM((n_chunks, chunk), jnp.int32),
            pltpu.VMEM((chunk, d), jnp.float32),
            pltpu.VMEM((chunk, d), jnp.float32),
            pltpu.VMEM((chunk, d), jnp.float32),
            pltpu.SemaphoreType.DMA,
            pltpu.SemaphoreType.DMA,
            pltpu.SemaphoreType.DMA,
            pltpu.SemaphoreType.DMA,
            pltpu.SemaphoreType.DMA,
            pltpu.SemaphoreType.DMA,
        ],
    )
    def k(src_hbm, idx_hbm, out_hbm, idx_v,
          buf0, buf1, buf2, l0, l1, l2, s0, s1, s2):
        wid = lax.axis_index("s") * _NUM_CORES + lax.axis_index("c")
        base = wid * (n_chunks * chunk)
        pltpu.sync_copy(idx_hbm.at[wid], idx_v)
        bufs = (buf0, buf1, buf2)
        lsems = (l0, l1, l2)
        ssems = (s0, s1, s2)

        def load(c, buf, sem):
            pltpu.async_copy(src_hbm.at[pl.ds(base + c * chunk, chunk)],
                             buf, sem)

        def scatter(c, buf, sem):
            pltpu.async_copy(buf, out_hbm.at[idx_v.at[c]], sem)

        def wait_load(buf, sem):
            pltpu.make_async_copy(src_hbm.at[pl.ds(0, chunk)], buf,
                                  sem).wait()

        def wait_scatter(buf, sem):
            pltpu.make_async_copy(buf, out_hbm.at[pl.ds(0, chunk)],
                                  sem).wait()

        load(0, bufs[0], lsems[0])
        load(1, bufs[1], lsems[1])
        for c in range(n_chunks):
            sl = c % 3
            if c >= 1 and c + 2 < n_chunks:
                sp = (c + 2) % 3
                wait_scatter(bufs[sp], ssems[sp])
                load(c + 2, bufs[sp], lsems[sp])
            elif c == 0 and n_chunks > 2:
                load(2, bufs[2], lsems[2])
            wait_load(bufs[sl], lsems[sl])
            scatter(c, bufs[sl], ssems[sl])
        for c in range(max(0, n_chunks - 3), n_chunks):
            sl = c % 3
            wait_scatter(bufs[sl], ssems[sl])

    return k(src, idx3)


def _sc_gather_rows(src, idx, n_out, chunk=_CHUNK):
    d = src.shape[1]
    rows_per_w = n_out // _NW
    n_chunks = rows_per_w // chunk
    assert rows_per_w % chunk == 0 and chunk <= 128 and chunk % 8 == 0
    assert n_chunks % 2 == 0 and n_chunks >= 2

    mesh = plsc.VectorSubcoreMesh(core_axis_name="c", subcore_axis_name="s")

    @functools.partial(
        pl.kernel,
        out_type=jax.ShapeDtypeStruct((n_out, d), jnp.float32),
        mesh=mesh,
        scratch_types=[
            pltpu.VMEM((rows_per_w,), jnp.int32),
            pltpu.VMEM((chunk, d), jnp.float32),
            pltpu.VMEM((chunk, d), jnp.float32),
            pltpu.VMEM((chunk, d), jnp.float32),
            pltpu.SemaphoreType.DMA,
            pltpu.SemaphoreType.DMA,
            pltpu.SemaphoreType.DMA,
            pltpu.SemaphoreType.DMA,
            pltpu.SemaphoreType.DMA,
            pltpu.SemaphoreType.DMA,
        ],
    )
    def k(src_hbm, idx_hbm, out_hbm, idx_v,
          buf0, buf1, buf2, g0, g1, g2, w0, w1, w2):
        wid = lax.axis_index("s") * _NUM_CORES + lax.axis_index("c")
        base = wid * rows_per_w
        pltpu.sync_copy(idx_hbm.at[pl.ds(base, rows_per_w)], idx_v)
        bufs = (buf0, buf1, buf2)
        gsems = (g0, g1, g2)
        wsems = (w0, w1, w2)

        def gather(c, buf, sem):
            pltpu.async_copy(src_hbm.at[idx_v.at[pl.ds(c * chunk, chunk)]],
                             buf, sem)

        def writeback(c, buf, sem):
            pltpu.async_copy(buf, out_hbm.at[pl.ds(base + c * chunk, chunk)],
                             sem)

        def wait_gather(buf, sem):
            pltpu.make_async_copy(src_hbm.at[pl.ds(0, chunk)], buf, sem).wait()

        def wait_writeback(buf, sem):
            pltpu.make_async_copy(buf, out_hbm.at[pl.ds(base, chunk)],
                                  sem).wait()

        gather(0, bufs[0], gsems[0])
        gather(1, bufs[1], gsems[1])
        for c in range(n_chunks):
            sl = c % 3
            if c >= 1 and c + 2 < n_chunks:
                sp = (c + 2) % 3
                wait_writeback(bufs[sp], wsems[sp])
                gather(c + 2, bufs[sp], gsems[sp])
            elif c == 0 and n_chunks > 2:
                gather(2, bufs[2], gsems[2])
            wait_gather(bufs[sl], gsems[sl])
            writeback(c, bufs[sl], wsems[sl])
        for c in range(max(0, n_chunks - 3), n_chunks):
            sl = c % 3
            wait_writeback(bufs[sl], wsems[sl])

    return k(src, idx)


def _grouped_matmul(x_pad, W, b, tile_expert):
    def mm_body(te_ref, x_ref, w_ref, b_ref, o_ref):
        o_ref[...] = (
            jnp.dot(x_ref[...], w_ref[0], preferred_element_type=jnp.float32)
            + b_ref[0]
        )

    grid_spec = pltpu.PrefetchScalarGridSpec(
        num_scalar_prefetch=1,
        grid=(N_TILES,),
        in_specs=[
            pl.BlockSpec((ROW_TILE, D_MODEL), lambda t, te: (t, 0)),
            pl.BlockSpec((1, D_MODEL, D_MODEL), lambda t, te: (te[t], 0, 0)),
            pl.BlockSpec((1, 1, D_MODEL), lambda t, te: (te[t], 0, 0)),
        ],
        out_specs=pl.BlockSpec((ROW_TILE, D_MODEL), lambda t, te: (t, 0)),
    )
    return pl.pallas_call(
        mm_body,
        grid_spec=grid_spec,
        out_shape=jax.ShapeDtypeStruct((N_PAD, D_MODEL), jnp.float32),
    )(tile_expert, x_pad, W, b.reshape(N_EXP, 1, D_MODEL))


def kernel(x, cell_dimensions, W, b):
    cd = cell_dimensions.astype(jnp.int32)

    onehot_t = (cd[None, :] == jnp.arange(N_EXP, dtype=jnp.int32)[:, None]
                ).astype(jnp.int32)
    cs = jnp.cumsum(onehot_t, axis=1)
    counts = cs[:, -1]
    padded = ((counts + ROW_TILE - 1) // ROW_TILE) * ROW_TILE
    padded_ends = jnp.cumsum(padded)
    offs_pad = padded_ends - padded
    dest_token = jnp.sum(onehot_t * (cs - 1 + offs_pad[:, None]), axis=0)
    dest_token = dest_token.astype(jnp.int32)
    tile_expert = jnp.minimum(
        (jnp.arange(N_TILES, dtype=jnp.int32)[:, None] * ROW_TILE
         >= padded_ends[None, :]).sum(1),
        N_EXP - 1).astype(jnp.int32)

    idx3 = dest_token.reshape(_NW, (N_TOK // _NW) // _CHUNK, _CHUNK)
    x_pad = _sc_scatter_rows(x, idx3, N_PAD)
    y_pad = _grouped_matmul(x_pad, W, b, tile_expert)
    out = _sc_gather_rows(y_pad, dest_token, N_TOK)
    return out
